# Initial kernel scaffold; baseline (speedup 1.0000x reference)
import functools

import jax
import jax.numpy as jnp
from jax import lax
from jax.experimental import pallas as pl
from jax.experimental.pallas import tpu as pltpu
from jax.experimental.pallas import tpu_sc as plsc

_L = 50
_D = 256
_B = 4096
_NW = 32
_ROWS_W = (_B * _L) // _NW
_IDX_COLS = 100
_IDX_ROWS_W = _ROWS_W // _IDX_COLS
_CHUNK = 200
_NCHUNK = _ROWS_W // _CHUNK
_GROUPS = _NCHUNK // 2
_LANES = 16


def _build():
    mesh = plsc.VectorSubcoreMesh(core_axis_name="c", subcore_axis_name="s")
    info = plsc.get_sparse_core_info()
    nc = info.num_cores

    @functools.partial(
        pl.kernel,
        mesh=mesh,
        out_type=jax.ShapeDtypeStruct((_B * _L, _D), jnp.float32),
        scratch_types=[
            pltpu.VMEM((_IDX_ROWS_W, _IDX_COLS), jnp.int32),
            pltpu.VMEM((_L, _D), jnp.float32),
            pltpu.VMEM((_CHUNK, _D), jnp.float32),
            pltpu.VMEM((_CHUNK, _D), jnp.float32),
            pltpu.SemaphoreType.DMA,
            pltpu.SemaphoreType.DMA,
            pltpu.SemaphoreType.DMA,
            pltpu.SemaphoreType.DMA,
        ],
    )
    def emb(x_hbm, tab_hbm, pos_hbm, out_hbm,
            idx_v, pos_v, buf0, buf1, sg0, sg1, ss0, ss1):
        wid = lax.axis_index("s") * nc + lax.axis_index("c")
        ibase = wid * _IDX_ROWS_W
        obase = wid * _ROWS_W

        pltpu.sync_copy(x_hbm.at[pl.ds(ibase, _IDX_ROWS_W)], idx_v)
        pltpu.sync_copy(pos_hbm, pos_v)

        bufs = (buf0, buf1)
        sgs = (sg0, sg1)
        sss = (ss0, ss1)

        def fire_gather(c, buf, sem):
            j0 = c * 2
            pltpu.async_copy(tab_hbm.at[idx_v.at[j0]],
                             buf.at[pl.ds(0, _IDX_COLS)], sem)
            pltpu.async_copy(tab_hbm.at[idx_v.at[j0 + 1]],
                             buf.at[pl.ds(_IDX_COLS, _IDX_COLS)], sem)

        def wait_gather(buf, sem):
            pltpu.make_async_copy(tab_hbm.at[pl.ds(0, _CHUNK)], buf, sem).wait()

        def add_pos(buf):
            def body(r, carry):
                pr = lax.rem(r, _L)
                for j in range(_D // _LANES):
                    sl = pl.ds(j * _LANES, _LANES)
                    buf[r, sl] = buf[r, sl] + pos_v[pr, sl]
                return carry
            lax.fori_loop(0, _CHUNK, body, 0)

        fire_gather(0, buf0, sg0)
        fire_gather(1, buf1, sg1)

        def group(g, carry):
            handles = []
            for b in range(2):
                c = 2 * g + b
                wait_gather(bufs[b], sgs[b])
                add_pos(bufs[b])
                handles.append(pltpu.async_copy(
                    bufs[b], out_hbm.at[pl.ds(obase + c * _CHUNK, _CHUNK)],
                    sss[b]))

            @pl.when(g < _GROUPS - 1)
            def _():
                for b in range(2):
                    handles[b].wait()
                    fire_gather(2 * g + 2 + b, bufs[b], sgs[b])

            return carry

        lax.fori_loop(0, _GROUPS, group, 0)

        for b in range(2):
            pltpu.make_async_copy(bufs[b], out_hbm.at[pl.ds(0, _CHUNK)],
                                  sss[b]).wait()

    return emb


_emb = _build()


def kernel(x, token_table, pos_table):
    xf = x.reshape(-1).astype(jnp.int32).reshape(_B * _L // _IDX_COLS, _IDX_COLS)
    out = _emb(xf, token_table, pos_table)
    return out.reshape(_B, _L, _D)


# speedup vs baseline: 1.4193x; 1.4193x over previous
import functools

import jax
import jax.numpy as jnp
from jax import lax
from jax.experimental import pallas as pl
from jax.experimental.pallas import tpu as pltpu
from jax.experimental.pallas import tpu_sc as plsc

_L = 50
_D = 256
_B = 4096
_NW = 32
_ROWS_W = (_B * _L) // _NW
_CHUNK = 128
_NCHUNK = _ROWS_W // _CHUNK
_GROUPS = _NCHUNK // 2
_LANES = 16


def _build():
    mesh = plsc.VectorSubcoreMesh(core_axis_name="c", subcore_axis_name="s")
    info = plsc.get_sparse_core_info()
    nc = info.num_cores

    @functools.partial(
        pl.kernel,
        mesh=mesh,
        out_type=jax.ShapeDtypeStruct((_B * _L, _D), jnp.float32),
        scratch_types=[
            pltpu.VMEM((_ROWS_W,), jnp.int32),
            pltpu.VMEM((_L, _D), jnp.float32),
            pltpu.VMEM((_CHUNK, _D), jnp.float32),
            pltpu.VMEM((_CHUNK, _D), jnp.float32),
            pltpu.SemaphoreType.DMA,
            pltpu.SemaphoreType.DMA,
            pltpu.SemaphoreType.DMA,
            pltpu.SemaphoreType.DMA,
        ],
    )
    def emb(x_hbm, tab_hbm, pos_hbm, out_hbm,
            idx_v, pos_v, buf0, buf1, sg0, sg1, ss0, ss1):
        wid = lax.axis_index("s") * nc + lax.axis_index("c")
        obase = wid * _ROWS_W

        pltpu.sync_copy(x_hbm.at[pl.ds(obase, _ROWS_W)], idx_v)
        pltpu.sync_copy(pos_hbm, pos_v)

        bufs = (buf0, buf1)
        sgs = (sg0, sg1)
        sss = (ss0, ss1)

        def fire_gather(c, buf, sem):
            pltpu.async_copy(tab_hbm.at[idx_v.at[pl.ds(c * _CHUNK, _CHUNK)]],
                             buf, sem)

        def wait_gather(buf, sem):
            pltpu.make_async_copy(tab_hbm.at[pl.ds(0, _CHUNK)], buf, sem).wait()

        def add_pos(c, buf):
            def body(r, carry):
                pr = lax.rem(c * _CHUNK + r, _L)
                for j in range(_D // _LANES):
                    sl = pl.ds(j * _LANES, _LANES)
                    buf[r, sl] = buf[r, sl] + pos_v[pr, sl]
                return carry
            lax.fori_loop(0, _CHUNK, body, 0)

        fire_gather(0, buf0, sg0)
        fire_gather(1, buf1, sg1)

        def group(g, carry):
            handles = []
            for b in range(2):
                c = 2 * g + b
                wait_gather(bufs[b], sgs[b])
                add_pos(c, bufs[b])
                handles.append(pltpu.async_copy(
                    bufs[b], out_hbm.at[pl.ds(obase + c * _CHUNK, _CHUNK)],
                    sss[b]))

            @pl.when(g < _GROUPS - 1)
            def _():
                for b in range(2):
                    handles[b].wait()
                    fire_gather(2 * g + 2 + b, bufs[b], sgs[b])

            return carry

        lax.fori_loop(0, _GROUPS, group, 0)

        for b in range(2):
            pltpu.make_async_copy(bufs[b], out_hbm.at[pl.ds(0, _CHUNK)],
                                  sss[b]).wait()

    return emb


_emb = _build()


def kernel(x, token_table, pos_table):
    xf = x.reshape(-1).astype(jnp.int32)
    out = _emb(xf, token_table, pos_table)
    return out.reshape(_B, _L, _D)


# speedup vs baseline: 1.5166x; 1.0685x over previous
import functools

import jax
import jax.numpy as jnp
from jax import lax
from jax.experimental import pallas as pl
from jax.experimental.pallas import tpu as pltpu
from jax.experimental.pallas import tpu_sc as plsc

_L = 50
_D = 256
_B = 4096
_NW = 32
_ROWS_W = (_B * _L) // _NW
_CHUNK = 80
_NCHUNK = _ROWS_W // _CHUNK
_GROUPS = _NCHUNK // 2
_LANES = 16


def _build():
    mesh = plsc.VectorSubcoreMesh(core_axis_name="c", subcore_axis_name="s")
    info = plsc.get_sparse_core_info()
    nc = info.num_cores

    @functools.partial(
        pl.kernel,
        mesh=mesh,
        out_type=jax.ShapeDtypeStruct((_B * _L, _D), jnp.float32),
        scratch_types=[
            pltpu.VMEM((_ROWS_W,), jnp.int32),
            pltpu.VMEM((_L, _D), jnp.float32),
            pltpu.VMEM((_CHUNK, _D), jnp.float32),
            pltpu.VMEM((_CHUNK, _D), jnp.float32),
            pltpu.VMEM((_CHUNK, _D), jnp.float32),
            pltpu.VMEM((_CHUNK, _D), jnp.float32),
            pltpu.SemaphoreType.DMA,
            pltpu.SemaphoreType.DMA,
            pltpu.SemaphoreType.DMA,
            pltpu.SemaphoreType.DMA,
        ],
    )
    def emb(x_hbm, tab_hbm, pos_hbm, out_hbm,
            idx_v, pos_v, gbuf0, gbuf1, sbuf0, sbuf1, sg0, sg1, ss0, ss1):
        wid = lax.axis_index("s") * nc + lax.axis_index("c")
        obase = wid * _ROWS_W

        pltpu.sync_copy(x_hbm.at[pl.ds(obase, _ROWS_W)], idx_v)
        pltpu.sync_copy(pos_hbm, pos_v)

        gbufs = (gbuf0, gbuf1)
        sbufs = (sbuf0, sbuf1)
        sgs = (sg0, sg1)
        sss = (ss0, ss1)

        def fire_gather(c, buf, sem):
            pltpu.async_copy(tab_hbm.at[idx_v.at[pl.ds(c * _CHUNK, _CHUNK)]],
                             buf, sem)

        def wait_gather(buf, sem):
            pltpu.make_async_copy(tab_hbm.at[pl.ds(0, _CHUNK)], buf, sem).wait()

        def wait_scatter(buf, sem):
            pltpu.make_async_copy(buf, out_hbm.at[pl.ds(0, _CHUNK)], sem).wait()

        def add_pos(c, gbuf, sbuf):
            def body(r, carry):
                pr = lax.rem(c * _CHUNK + r, _L)
                for j in range(_D // _LANES):
                    sl = pl.ds(j * _LANES, _LANES)
                    sbuf[r, sl] = gbuf[r, sl] + pos_v[pr, sl]
                return carry
            lax.fori_loop(0, _CHUNK, body, 0)

        fire_gather(0, gbuf0, sg0)
        fire_gather(1, gbuf1, sg1)

        def group(g, carry):
            for b in range(2):
                c = 2 * g + b
                wait_gather(gbufs[b], sgs[b])

                @pl.when(g > 0)
                def _():
                    wait_scatter(sbufs[b], sss[b])

                add_pos(c, gbufs[b], sbufs[b])

                @pl.when(g < _GROUPS - 1)
                def _():
                    fire_gather(c + 2, gbufs[b], sgs[b])

                pltpu.async_copy(
                    sbufs[b], out_hbm.at[pl.ds(obase + c * _CHUNK, _CHUNK)],
                    sss[b])
            return carry

        lax.fori_loop(0, _GROUPS, group, 0)

        for b in range(2):
            wait_scatter(sbufs[b], sss[b])

    return emb


_emb = _build()


def kernel(x, token_table, pos_table):
    out = _emb(x.reshape(-1).astype(jnp.int32), token_table, pos_table)
    return out.reshape(_B, _L, _D)


# speedup vs baseline: 2.4131x; 1.5911x over previous
import functools

import jax
import jax.numpy as jnp
from jax import lax
from jax.experimental import pallas as pl
from jax.experimental.pallas import tpu as pltpu
from jax.experimental.pallas import tpu_sc as plsc

_L = 50
_D = 256
_B = 4096
_NW = 32
_ROWS_W = (_B * _L) // _NW
_CHUNK = 80
_NCHUNK = _ROWS_W // _CHUNK
_GROUPS = _NCHUNK // 2
_LANES = 16


def _build():
    mesh = plsc.VectorSubcoreMesh(core_axis_name="c", subcore_axis_name="s")
    info = plsc.get_sparse_core_info()
    nc = info.num_cores

    @functools.partial(
        pl.kernel,
        mesh=mesh,
        out_type=jax.ShapeDtypeStruct((_B * _L, _D), jnp.float32),
        scratch_types=[
            pltpu.VMEM((_ROWS_W,), jnp.int32),
            pltpu.VMEM((_L, _D), jnp.float32),
            pltpu.VMEM((_CHUNK, _D), jnp.float32),
            pltpu.VMEM((_CHUNK, _D), jnp.float32),
            pltpu.VMEM((_CHUNK, _D), jnp.float32),
            pltpu.VMEM((_CHUNK, _D), jnp.float32),
            pltpu.SemaphoreType.DMA,
            pltpu.SemaphoreType.DMA,
            pltpu.SemaphoreType.DMA,
            pltpu.SemaphoreType.DMA,
        ],
    )
    def emb(x_hbm, tab_hbm, pos_hbm, out_hbm,
            idx_v, pos_v, gbuf0, gbuf1, sbuf0, sbuf1, sg0, sg1, ss0, ss1):
        wid = lax.axis_index("s") * nc + lax.axis_index("c")
        obase = wid * _ROWS_W

        pltpu.sync_copy(x_hbm.at[pl.ds(obase, _ROWS_W)], idx_v)
        pltpu.sync_copy(pos_hbm, pos_v)

        gbufs = (gbuf0, gbuf1)
        sbufs = (sbuf0, sbuf1)
        sgs = (sg0, sg1)
        sss = (ss0, ss1)

        def fire_gather(c, buf, sem):
            pltpu.async_copy(tab_hbm.at[idx_v.at[pl.ds(c * _CHUNK, _CHUNK)]],
                             buf, sem)

        def wait_gather(buf, sem):
            pltpu.make_async_copy(tab_hbm.at[pl.ds(0, _CHUNK)], buf, sem).wait()

        def wait_scatter(buf, sem):
            pltpu.make_async_copy(buf, out_hbm.at[pl.ds(0, _CHUNK)], sem).wait()

        def add_pos(c, gbuf, sbuf):
            def body(r, carry):
                pr = lax.rem(c * _CHUNK + r, _L)
                nj = _D // _LANES
                sls = [pl.ds(j * _LANES, _LANES) for j in range(nj)]
                gv = [gbuf[r, sls[j]] for j in range(nj)]
                pv = [pos_v[pr, sls[j]] for j in range(nj)]
                for j in range(nj):
                    sbuf[r, sls[j]] = gv[j] + pv[j]
                return carry
            lax.fori_loop(0, _CHUNK, body, 0)

        fire_gather(0, gbuf0, sg0)
        fire_gather(1, gbuf1, sg1)

        def group(g, carry):
            for b in range(2):
                c = 2 * g + b
                wait_gather(gbufs[b], sgs[b])

                @pl.when(g > 0)
                def _():
                    wait_scatter(sbufs[b], sss[b])

                add_pos(c, gbufs[b], sbufs[b])

                @pl.when(g < _GROUPS - 1)
                def _():
                    fire_gather(c + 2, gbufs[b], sgs[b])

                pltpu.async_copy(
                    sbufs[b], out_hbm.at[pl.ds(obase + c * _CHUNK, _CHUNK)],
                    sss[b])
            return carry

        lax.fori_loop(0, _GROUPS, group, 0)

        for b in range(2):
            wait_scatter(sbufs[b], sss[b])

    return emb


_emb = _build()


def kernel(x, token_table, pos_table):
    out = _emb(x.reshape(-1).astype(jnp.int32), token_table, pos_table)
    return out.reshape(_B, _L, _D)


# speedup vs baseline: 3.7346x; 1.5477x over previous
import functools

import jax
import jax.numpy as jnp
from jax import lax
from jax.experimental import pallas as pl
from jax.experimental.pallas import tpu as pltpu
from jax.experimental.pallas import tpu_sc as plsc

_L = 50
_D = 256
_B = 4096
_NW = 32
_ROWS_W = (_B * _L) // _NW
_SEQ_W = _B // _NW
_CHUNK = 80
_NCHUNK = _ROWS_W // _CHUNK
_SUPER = 10
_NSUPER = _NCHUNK // _SUPER
_PLANES = 4
_LANES = 16

_SCATTER_AT = {0: (0,), 1: (1, 2), 2: (3,), 3: (4, 5), 4: (6, 7),
               5: (8,), 6: (9, 10), 7: (11,), 8: (12, 13), 9: (14, 15)}
_ENTER_AT = {0: (0, 1), 1: (2, 3), 2: (4,), 3: (5, 6), 4: (7,),
             5: (8, 9), 6: (10, 11), 7: (12,), 8: (13, 14), 9: (15,)}


def _build():
    mesh = plsc.VectorSubcoreMesh(core_axis_name="c", subcore_axis_name="s")
    info = plsc.get_sparse_core_info()
    nc = info.num_cores

    @functools.partial(
        pl.kernel,
        mesh=mesh,
        out_type=jax.ShapeDtypeStruct((_B, _L, _D), jnp.float32),
        scratch_types=[
            pltpu.VMEM((_ROWS_W,), jnp.int32),
            pltpu.VMEM((_L, _D), jnp.float32),
            pltpu.VMEM((_CHUNK, _D), jnp.float32),
            pltpu.VMEM((_CHUNK, _D), jnp.float32),
            pltpu.VMEM((_PLANES, _L, _D), jnp.float32),
            pltpu.SemaphoreType.DMA,
            pltpu.SemaphoreType.DMA,
            pltpu.SemaphoreType.DMA,
            pltpu.SemaphoreType.DMA,
            pltpu.SemaphoreType.DMA,
            pltpu.SemaphoreType.DMA,
        ],
    )
    def emb(x_hbm, tab_hbm, pos_hbm, out_hbm,
            idx_v, pos_v, gbuf0, gbuf1, sbuf,
            sg0, sg1, sp0, sp1, sp2, sp3):
        wid = lax.axis_index("s") * nc + lax.axis_index("c")
        rbase = wid * _ROWS_W
        seq_base = wid * _SEQ_W

        pltpu.sync_copy(x_hbm.at[pl.ds(rbase, _ROWS_W)], idx_v)
        pltpu.sync_copy(pos_hbm, pos_v)

        gbufs = (gbuf0, gbuf1)
        sgs = (sg0, sg1)
        sps = (sp0, sp1, sp2, sp3)

        def fire_gather(c, buf, sem):
            pltpu.async_copy(tab_hbm.at[idx_v.at[pl.ds(c * _CHUNK, _CHUNK)]],
                             buf, sem)

        def wait_gather(buf, sem):
            pltpu.make_async_copy(tab_hbm.at[pl.ds(0, _CHUNK)], buf, sem).wait()

        def wait_plane(p):
            pltpu.make_async_copy(sbuf.at[p], out_hbm.at[0], sps[p]).wait()

        def add_chunk(g, i, gbuf):
            def body(r, carry):
                t = g * (_SUPER * _CHUNK) + i * _CHUNK + r
                sq = lax.div(t, _L)
                pr = lax.rem(t, _L)
                pq = lax.rem(sq, _PLANES)
                nj = _D // _LANES
                sls = [pl.ds(j * _LANES, _LANES) for j in range(nj)]
                gv = [gbuf[r, sls[j]] for j in range(nj)]
                pv = [pos_v[pr, sls[j]] for j in range(nj)]
                for j in range(nj):
                    sbuf[pq, pr, sls[j]] = gv[j] + pv[j]
                return carry
            lax.fori_loop(0, _CHUNK, body, 0)

        fire_gather(0, gbuf0, sg0)
        fire_gather(1, gbuf1, sg1)

        def group(g, carry):
            for i in range(_SUPER):
                c = g * _SUPER + i
                for s in _ENTER_AT[i]:
                    if s < _PLANES:
                        @pl.when(g > 0)
                        def _(p=s % _PLANES):
                            wait_plane(p)
                    else:
                        wait_plane(s % _PLANES)

                wait_gather(gbufs[i % 2], sgs[i % 2])
                add_chunk(g, i, gbufs[i % 2])

                @pl.when(c + 2 < _NCHUNK)
                def _(b=i % 2, cc=c):
                    fire_gather(cc + 2, gbufs[b], sgs[b])

                for s in _SCATTER_AT[i]:
                    p = s % _PLANES
                    seqg = seq_base + g * (_SUPER * _CHUNK // _L) + s
                    pltpu.async_copy(sbuf.at[p], out_hbm.at[seqg], sps[p])
            return carry

        lax.fori_loop(0, _NSUPER, group, 0)

        for p in range(_PLANES):
            wait_plane(p)

    return emb


_emb = _build()


def kernel(x, token_table, pos_table):
    return _emb(x.reshape(-1).astype(jnp.int32), token_table, pos_table)


# speedup vs baseline: 7.6838x; 2.0575x over previous
import functools

import jax
import jax.numpy as jnp
from jax import lax
from jax.experimental import pallas as pl
from jax.experimental.pallas import tpu as pltpu
from jax.experimental.pallas import tpu_sc as plsc

_L = 50
_D = 256
_B = 4096
_NW = 32
_BW = _B // _NW
_ROWS_W = _L * _BW
_RING = 3
_MAIN = 48
_LANES = 16


def _build():
    mesh = plsc.VectorSubcoreMesh(core_axis_name="c", subcore_axis_name="s")
    info = plsc.get_sparse_core_info()
    nc = info.num_cores

    @functools.partial(
        pl.kernel,
        mesh=mesh,
        out_type=jax.ShapeDtypeStruct((_L, _B, _D), jnp.float32),
        scratch_types=[
            pltpu.VMEM((_ROWS_W,), jnp.int32),
            pltpu.VMEM((_L, _D), jnp.float32),
            pltpu.VMEM((_BW, _D), jnp.float32),
            pltpu.VMEM((_BW, _D), jnp.float32),
            pltpu.VMEM((_BW, _D), jnp.float32),
            pltpu.SemaphoreType.DMA,
            pltpu.SemaphoreType.DMA,
            pltpu.SemaphoreType.DMA,
            pltpu.SemaphoreType.DMA,
            pltpu.SemaphoreType.DMA,
            pltpu.SemaphoreType.DMA,
        ],
    )
    def emb(x_hbm, tab_hbm, pos_hbm, out_hbm,
            idx_v, pos_v, buf0, buf1, buf2,
            sg0, sg1, sg2, ss0, ss1, ss2):
        wid = lax.axis_index("s") * nc + lax.axis_index("c")
        bbase = wid * _BW

        pltpu.sync_copy(x_hbm.at[pl.ds(wid * _ROWS_W, _ROWS_W)], idx_v)
        pltpu.sync_copy(pos_hbm, pos_v)

        bufs = (buf0, buf1, buf2)
        sgs = (sg0, sg1, sg2)
        sss = (ss0, ss1, ss2)

        def fire_gather(t, j):
            pltpu.async_copy(tab_hbm.at[idx_v.at[pl.ds(t * _BW, _BW)]],
                             bufs[j], sgs[j])

        def wait_gather(j):
            pltpu.make_async_copy(tab_hbm.at[pl.ds(0, _BW)], bufs[j],
                                  sgs[j]).wait()

        def wait_scatter(j):
            pltpu.make_async_copy(bufs[j], out_hbm.at[0, pl.ds(0, _BW)],
                                  sss[j]).wait()

        def add_pos(t, j):
            buf = bufs[j]
            nj = _D // _LANES
            sls = [pl.ds(k * _LANES, _LANES) for k in range(nj)]
            pv = [pos_v[t, sls[k]] for k in range(nj)]

            def body(r, carry):
                gv = [buf[r, sls[k]] for k in range(nj)]
                for k in range(nj):
                    buf[r, sls[k]] = gv[k] + pv[k]
                return carry
            lax.fori_loop(0, _BW, body, 0)

        def step(u, i, first_round):
            j = i % _RING
            jp = (i + 2) % _RING
            wait_gather(j)
            add_pos(u, j)
            pltpu.async_copy(bufs[j], out_hbm.at[u, pl.ds(bbase, _BW)],
                             sss[j])
            if isinstance(u, int):
                if u + 2 < _L:
                    if not first_round:
                        wait_scatter(jp)
                    fire_gather(u + 2, jp)
            else:
                @pl.when(u + 2 < _L)
                def _():
                    wait_scatter(jp)
                    fire_gather(u + 2, jp)

        fire_gather(0, 0)
        fire_gather(1, 1)

        step(0, 0, True)
        step(1, 1, False)
        step(2, 2, False)

        def group(g, carry):
            for i in range(_RING):
                step(3 + 3 * g + i, i, False)
            return carry

        lax.fori_loop(0, (_MAIN - 3) // _RING, group, 0)

        step(48, 0, False)
        step(49, 1, False)

        for j in (2, 0, 1):
            wait_scatter(j)

    return emb


_emb = _build()


def kernel(x, token_table, pos_table):
    xp = x.astype(jnp.int32).T.reshape(_L, _NW, _BW)
    xp = xp.transpose(1, 0, 2).reshape(-1)
    out = _emb(xp, token_table, pos_table)
    return out.transpose(1, 0, 2)


# speedup vs baseline: 7.7287x; 1.0058x over previous
import functools

import jax
import jax.numpy as jnp
from jax import lax
from jax.experimental import pallas as pl
from jax.experimental.pallas import tpu as pltpu
from jax.experimental.pallas import tpu_sc as plsc

_L = 50
_D = 256
_B = 4096
_NW = 32
_BW = _B // _NW
_HB = _BW // 2
_ROWS_W = _L * _BW
_STEPS = 2 * _L
_RING = 5
_LANES = 16


def _build():
    mesh = plsc.VectorSubcoreMesh(core_axis_name="c", subcore_axis_name="s")
    info = plsc.get_sparse_core_info()
    nc = info.num_cores

    @functools.partial(
        pl.kernel,
        mesh=mesh,
        out_type=jax.ShapeDtypeStruct((_L, _B, _D), jnp.float32),
        scratch_types=[
            pltpu.VMEM((_ROWS_W,), jnp.int32),
            pltpu.VMEM((_L, _D), jnp.float32),
            pltpu.VMEM((_HB, _D), jnp.float32),
            pltpu.VMEM((_HB, _D), jnp.float32),
            pltpu.VMEM((_HB, _D), jnp.float32),
            pltpu.VMEM((_HB, _D), jnp.float32),
            pltpu.VMEM((_HB, _D), jnp.float32),
            pltpu.SemaphoreType.DMA,
            pltpu.SemaphoreType.DMA,
            pltpu.SemaphoreType.DMA,
            pltpu.SemaphoreType.DMA,
            pltpu.SemaphoreType.DMA,
            pltpu.SemaphoreType.DMA,
            pltpu.SemaphoreType.DMA,
            pltpu.SemaphoreType.DMA,
            pltpu.SemaphoreType.DMA,
            pltpu.SemaphoreType.DMA,
        ],
    )
    def emb(x_hbm, tab_hbm, pos_hbm, out_hbm,
            idx_v, pos_v, buf0, buf1, buf2, buf3, buf4,
            sg0, sg1, sg2, sg3, sg4, ss0, ss1, ss2, ss3, ss4):
        wid = lax.axis_index("s") * nc + lax.axis_index("c")
        bbase = wid * _BW

        pltpu.sync_copy(x_hbm.at[pl.ds(wid * _ROWS_W, _ROWS_W)], idx_v)
        pltpu.sync_copy(pos_hbm, pos_v)

        bufs = (buf0, buf1, buf2, buf3, buf4)
        sgs = (sg0, sg1, sg2, sg3, sg4)
        sss = (ss0, ss1, ss2, ss3, ss4)

        def fire_gather(u, j):
            pltpu.async_copy(tab_hbm.at[idx_v.at[pl.ds(u * _HB, _HB)]],
                             bufs[j], sgs[j])

        def wait_gather(j):
            pltpu.make_async_copy(tab_hbm.at[pl.ds(0, _HB)], bufs[j],
                                  sgs[j]).wait()

        def wait_scatter(j):
            pltpu.make_async_copy(bufs[j], out_hbm.at[0, pl.ds(0, _HB)],
                                  sss[j]).wait()

        def add_pos(t, j):
            buf = bufs[j]
            nj = _D // _LANES
            sls = [pl.ds(k * _LANES, _LANES) for k in range(nj)]
            pv = [pos_v[t, sls[k]] for k in range(nj)]

            def body(r, carry):
                gv = [buf[r, sls[k]] for k in range(nj)]
                for k in range(nj):
                    buf[r, sls[k]] = gv[k] + pv[k]
                return carry
            lax.fori_loop(0, _HB, body, 0)

        def step(u, i):
            j = i % _RING
            jp = (i + 3) % _RING
            t = u // 2 if isinstance(u, int) else lax.div(u, 2)
            half = u % 2 if isinstance(u, int) else lax.rem(u, 2)
            wait_gather(j)
            add_pos(t, j)
            pltpu.async_copy(
                bufs[j], out_hbm.at[t, pl.ds(bbase + half * _HB, _HB)],
                sss[j])
            if isinstance(u, int):
                if u + 3 < _STEPS:
                    if u >= 2:
                        wait_scatter(jp)
                    fire_gather(u + 3, jp)
            else:
                @pl.when(u + 3 < _STEPS)
                def _():
                    wait_scatter(jp)
                    fire_gather(u + 3, jp)

        fire_gather(0, 0)
        fire_gather(1, 1)
        fire_gather(2, 2)

        for u in range(_RING):
            step(u, u)

        def group(g, carry):
            for i in range(_RING):
                step(_RING + _RING * g + i, i)
            return carry

        lax.fori_loop(0, (_STEPS - 2 * _RING) // _RING, group, 0)

        for u in range(_STEPS - _RING, _STEPS):
            step(u, u % _RING)

        for u in range(_STEPS - _RING, _STEPS):
            wait_scatter(u % _RING)

    return emb


_emb = _build()


def kernel(x, token_table, pos_table):
    xp = x.astype(jnp.int32).T.reshape(_L, _NW, _BW)
    xp = xp.transpose(1, 0, 2).reshape(-1)
    out = _emb(xp, token_table, pos_table)
    return out.transpose(1, 0, 2)
